# baseline (speedup 1.0000x reference)
import functools

import jax
import jax.numpy as jnp
from jax import lax
from jax.experimental import pallas as pl
from jax.experimental.pallas import tpu as pltpu
from jax.experimental.pallas import tpu_sc as plsc

_T = 100
_H = 4096
_E = 64
_K = 8
_GAMMA = 0.5
_EPS = 1e-6
_B = 4
_S = 2048

_CS = 256
_NSTEP = _S // _CS
_ROWS = _B * _T
_NWK = 32
_RP = 416


def _routing_body(hb_ref, hbase_ref, wproj_ref, idx_ref, w_ref, acc_ref):
    i = pl.program_id(0)

    @pl.when(i == 0)
    def _init():
        acc_ref[...] = jnp.zeros_like(acc_ref)

    for b in range(_B):
        acc_ref[b : b + 1, :] += jnp.sum(hb_ref[b], axis=0, keepdims=True)

    @pl.when(i == _NSTEP - 1)
    def _finish():
        pooled = acc_ref[...] * (1.0 / _S)
        delta = jnp.dot(pooled, wproj_ref[...],
                        preferred_element_type=jnp.float32)
        hbase = hbase_ref[...]
        h_scale = jnp.maximum(jnp.max(jnp.abs(hbase)), _EPS)
        d_scale = jnp.maximum(jnp.max(jnp.abs(delta)), _EPS)
        hn = hbase * ((1.0 - _GAMMA) / h_scale)
        dn = delta * (_GAMMA / d_scale)
        iota_e = lax.broadcasted_iota(jnp.int32, (_T, _E), 1)
        for b in range(_B):
            logits = hn + dn[b : b + 1, :]
            p = jnp.exp(logits - jnp.max(logits, axis=1, keepdims=True))
            sel = jnp.zeros((_T, _E), dtype=jnp.bool_)
            wcols, icols = [], []
            for _k in range(_K):
                cur = jnp.where(sel, -1.0, p)
                m = jnp.max(cur, axis=1, keepdims=True)
                cand = jnp.where(cur == m, iota_e, _E)
                amin = jnp.min(cand, axis=1, keepdims=True)
                sel = sel | (iota_e == amin)
                wcols.append(m)
                icols.append(amin)
            w = jnp.concatenate(wcols, axis=1)
            idx = jnp.concatenate(icols, axis=1)
            w = w / jnp.maximum(jnp.sum(w, axis=1, keepdims=True), 1e-9)
            idx_ref[pl.ds(b * _T, _T), :] = idx
            w_ref[pl.ds(b * _T, _T), :] = w
        idx_ref[pl.ds(_ROWS, _RP - _ROWS), :] = jnp.zeros(
            (_RP - _ROWS, _K), jnp.int32)
        w_ref[pl.ds(_ROWS, _RP - _ROWS), :] = jnp.zeros(
            (_RP - _ROWS, _K), jnp.float32)


def _routing(hidden_states, H_base, W_proj):
    return pl.pallas_call(
        _routing_body,
        grid=(_NSTEP,),
        in_specs=[
            pl.BlockSpec((_B, _CS, _H), lambda i: (0, i, 0)),
            pl.BlockSpec((_T, _E), lambda i: (0, 0)),
            pl.BlockSpec((_H, _E), lambda i: (0, 0)),
        ],
        out_specs=[
            pl.BlockSpec((_RP, _K), lambda i: (0, 0)),
            pl.BlockSpec((_RP, _K), lambda i: (0, 0)),
        ],
        out_shape=[
            jax.ShapeDtypeStruct((_RP, _K), jnp.int32),
            jax.ShapeDtypeStruct((_RP, _K), jnp.float32),
        ],
        scratch_shapes=[pltpu.VMEM((_B, _H), jnp.float32)],
        compiler_params=pltpu.CompilerParams(
            dimension_semantics=("arbitrary",)),
    )(hidden_states, H_base, W_proj)


def _sc_mix(LiMEs, idx, w):
    info = plsc.get_sparse_core_info()
    ncores, nsub, lanes = info.num_cores, info.num_subcores, info.num_lanes
    nwk = ncores * nsub
    cw = _H // nwk
    nchunk = cw // lanes
    mesh = plsc.VectorSubcoreMesh(core_axis_name="c", subcore_axis_name="s")

    @functools.partial(
        pl.kernel,
        mesh=mesh,
        compiler_params=pltpu.CompilerParams(needs_layout_passes=False),
        out_type=jax.ShapeDtypeStruct((_RP, _H), jnp.float32),
        scratch_types=[
            pltpu.VMEM((_E * cw,), jnp.float32),
            pltpu.VMEM((_RP * _K,), jnp.int32),
            pltpu.VMEM((_RP * _K,), jnp.float32),
            pltpu.VMEM((_RP, cw), jnp.float32),
        ],
    )
    def mix(limes_hbm, idx_hbm, w_hbm, out_hbm, tbl_v, idx_v, w_v, obuf_v):
        wid = lax.axis_index("s") * ncores + lax.axis_index("c")
        col0 = wid * cw
        for e in range(_E):
            pltpu.sync_copy(limes_hbm.at[e, pl.ds(col0, cw)],
                            tbl_v.at[pl.ds(e * cw, cw)])
        pltpu.sync_copy(idx_hbm, idx_v)
        pltpu.sync_copy(w_hbm, w_v)
        lane_iota = lax.iota(jnp.int32, lanes)

        def row_body(r, carry):
            rsp = jnp.zeros((lanes,), jnp.int32) + (r * _K)
            pairs = []
            for k in range(_K):
                ksp = jnp.full((lanes,), k, jnp.int32)
                ik = plsc.load_gather(idx_v, [rsp + ksp])
                wk = plsc.load_gather(w_v, [rsp + ksp])
                pairs.append((ik * cw, wk))
            for c in range(nchunk):
                colv = lane_iota + (c * lanes)
                acc = jnp.zeros((lanes,), jnp.float32)
                for ik, wk in pairs:
                    acc = acc + wk * plsc.load_gather(tbl_v, [ik + colv])
                obuf_v[r, pl.ds(c * lanes, lanes)] = acc
            return carry

        lax.fori_loop(0, _RP, row_body, 0)
        pltpu.sync_copy(obuf_v, out_hbm.at[:, pl.ds(col0, cw)])

    return mix(LiMEs, idx.reshape(-1), w.reshape(-1))


def kernel(hidden_states, LiMEs, H_base, W_proj):
    idx, w = _routing(hidden_states, H_base, W_proj)
    out = _sc_mix(LiMEs, idx, w)
    return out[:_ROWS].reshape(_B, _T, _H)
